# Initial kernel scaffold; baseline (speedup 1.0000x reference)
import jax
import jax.numpy as jnp
from jax.experimental import pallas as pl


def kernel(x, w0, b0, g0, be0, w1, b1, g1, be1, w2, b2, g2, be2, w3, b3, g3, be3):
    raise NotImplementedError("write your pallas kernel here")


# speedup vs baseline: 3.3363x; 3.3363x over previous
import functools

import jax
import jax.numpy as jnp
from jax.experimental import pallas as pl
from jax.experimental.pallas import tpu as pltpu

_EPS = 1e-5


def _pick_tm(n):
    for t in (2048, 1024, 512, 256, 128, 64, 32, 16, 8):
        if n % t == 0:
            return t
    return n


def _acc_stats(st_ref, z, i):
    @pl.when(i == 0)
    def _():
        st_ref[...] = jnp.zeros_like(st_ref)

    ones = jnp.ones((1, z.shape[0]), jnp.float32)
    st_ref[0:1, :] += jnp.dot(ones, z, preferred_element_type=jnp.float32)
    st_ref[1:2, :] += jnp.dot(ones, z * z, preferred_element_type=jnp.float32)


def _fold(st_ref, g_ref, be_ref, ss_ref, inv_n):
    mu = st_ref[0:1, :] * inv_n
    var = st_ref[1:2, :] * inv_n - mu * mu
    scale = g_ref[...] * jax.lax.rsqrt(var + _EPS)
    ss_ref[0:1, :] = scale
    ss_ref[1:2, :] = be_ref[...] - mu * scale


def _bn_relu(z, ss_ref):
    h = jnp.maximum(z * ss_ref[0:1, :] + ss_ref[1:2, :], 0.0)
    return h.astype(jnp.bfloat16)


def _mlp_bn_body(tm, nb, inv_n,
                 x_ref, w0_ref, w1_ref, w2_ref, w3_ref,
                 g0_ref, be0_ref, g1_ref, be1_ref,
                 g2_ref, be2_ref, g3_ref, be3_ref,
                 o_ref,
                 z3_buf, st1, st2, st3, st4, ss1, ss2, ss3, ss4):
    p = pl.program_id(0)
    i = pl.program_id(1)

    def z1_of_x():
        return jnp.dot(x_ref[...], w0_ref[...],
                       preferred_element_type=jnp.float32)

    def z4_of_cache():
        z3 = z3_buf[pl.ds(i * tm, tm), :].astype(jnp.float32)
        h3 = _bn_relu(z3, ss3)
        return jnp.dot(h3, w3_ref[...], preferred_element_type=jnp.float32)

    @pl.when(p == 0)
    def _():
        z1 = z1_of_x()
        _acc_stats(st1, z1, i)

        @pl.when(i == nb - 1)
        def _():
            _fold(st1, g0_ref, be0_ref, ss1, inv_n)

    @pl.when(p == 1)
    def _():
        h1 = _bn_relu(z1_of_x(), ss1)
        z2 = jnp.dot(h1, w1_ref[...], preferred_element_type=jnp.float32)
        _acc_stats(st2, z2, i)

        @pl.when(i == nb - 1)
        def _():
            _fold(st2, g1_ref, be1_ref, ss2, inv_n)

    @pl.when(p == 2)
    def _():
        h1 = _bn_relu(z1_of_x(), ss1)
        z2 = jnp.dot(h1, w1_ref[...], preferred_element_type=jnp.float32)
        h2 = _bn_relu(z2, ss2)
        z3 = jnp.dot(h2, w2_ref[...], preferred_element_type=jnp.float32)
        _acc_stats(st3, z3, i)
        z3_buf[pl.ds(i * tm, tm), :] = z3.astype(jnp.bfloat16)

        @pl.when(i == nb - 1)
        def _():
            _fold(st3, g2_ref, be2_ref, ss3, inv_n)

    @pl.when(p == 3)
    def _():
        z4 = z4_of_cache()
        _acc_stats(st4, z4, i)

        @pl.when(i == nb - 1)
        def _():
            _fold(st4, g3_ref, be3_ref, ss4, inv_n)

    @pl.when(p == 4)
    def _():
        z4 = z4_of_cache()
        o_ref[...] = z4 * ss4[0:1, :] + ss4[1:2, :]


def kernel(x, w0, b0, g0, be0, w1, b1, g1, be1, w2, b2, g2, be2,
           w3, b3, g3, be3):
    n, f_in = x.shape
    f1, f2, f3, f4 = w0.shape[1], w1.shape[1], w2.shape[1], w3.shape[1]
    tm = _pick_tm(n)
    nb = n // tm

    xb = x.astype(jnp.bfloat16)
    w0b, w1b, w2b, w3b = (w.astype(jnp.bfloat16) for w in (w0, w1, w2, w3))

    fixed = lambda p, i: (0, 0)
    body = functools.partial(_mlp_bn_body, tm, nb, 1.0 / n)

    return pl.pallas_call(
        body,
        out_shape=jax.ShapeDtypeStruct((n, f4), jnp.float32),
        grid=(5, nb),
        in_specs=[
            pl.BlockSpec((tm, f_in), lambda p, i: (jnp.where(p < 3, i, 0), 0)),
            pl.BlockSpec((f_in, f1), fixed),
            pl.BlockSpec((f1, f2), fixed),
            pl.BlockSpec((f2, f3), fixed),
            pl.BlockSpec((f3, f4), fixed),
            pl.BlockSpec((1, f1), fixed), pl.BlockSpec((1, f1), fixed),
            pl.BlockSpec((1, f2), fixed), pl.BlockSpec((1, f2), fixed),
            pl.BlockSpec((1, f3), fixed), pl.BlockSpec((1, f3), fixed),
            pl.BlockSpec((1, f4), fixed), pl.BlockSpec((1, f4), fixed),
        ],
        out_specs=pl.BlockSpec((tm, f4),
                               lambda p, i: (jnp.where(p == 4, i, 0), 0)),
        scratch_shapes=[
            pltpu.VMEM((n, f3), jnp.bfloat16),
            pltpu.VMEM((2, f1), jnp.float32),
            pltpu.VMEM((2, f2), jnp.float32),
            pltpu.VMEM((2, f3), jnp.float32),
            pltpu.VMEM((2, f4), jnp.float32),
            pltpu.VMEM((2, f1), jnp.float32),
            pltpu.VMEM((2, f2), jnp.float32),
            pltpu.VMEM((2, f3), jnp.float32),
            pltpu.VMEM((2, f4), jnp.float32),
        ],
        compiler_params=pltpu.CompilerParams(
            dimension_semantics=("arbitrary", "arbitrary"),
            vmem_limit_bytes=52 * 1024 * 1024,
        ),
        name="mlp_bn4_fused",
    )(xb, w0b, w1b, w2b, w3b, g0, be0, g1, be1, g2, be2, g3, be3)


# speedup vs baseline: 3.4166x; 1.0241x over previous
import functools

import jax
import jax.numpy as jnp
from jax.experimental import pallas as pl
from jax.experimental.pallas import tpu as pltpu

_EPS = 1e-5
_ROW_CHUNK = 2048


def _pick_tm(n):
    for t in (4096, 2048, 1024, 512, 256, 128, 64, 32, 16, 8):
        if n % t == 0:
            return t
    return n


def _acc_stats(st_ref, z, i):
    @pl.when(i == 0)
    def _():
        st_ref[...] = jnp.zeros_like(st_ref)

    ones = jnp.ones((1, z.shape[0]), jnp.float32)
    st_ref[0:1, :] += jnp.dot(ones, z, preferred_element_type=jnp.float32)
    st_ref[1:2, :] += jnp.dot(ones, z * z, preferred_element_type=jnp.float32)


def _fold(st_ref, g_ref, be_ref, ss_ref, inv_n):
    mu = st_ref[0:1, :] * inv_n
    var = st_ref[1:2, :] * inv_n - mu * mu
    scale = g_ref[...] * jax.lax.rsqrt(var + _EPS)
    ss_ref[0:1, :] = scale
    ss_ref[1:2, :] = be_ref[...] - mu * scale


def _store_rows(dst_ref, base, vals_bf16):
    tm = vals_bf16.shape[0]
    for r0 in range(0, tm, _ROW_CHUNK):
        r1 = min(r0 + _ROW_CHUNK, tm)
        dst_ref[pl.ds(base + r0, r1 - r0), :] = vals_bf16[r0:r1, :]


def _mlp_bn_body(tm, nb, inv_n,
                 xa_ref, w0a_ref, w1_ref, w2_ref, w3_ref,
                 g0_ref, be0_ref, g1_ref, be1_ref,
                 g2_ref, be2_ref, g3_ref, be3_ref,
                 o_ref,
                 z3c, w0f, w1f, st1, st2, st3, st4, ss1, ss2, ss3, ss4):
    p = pl.program_id(0)
    i = pl.program_id(1)

    def h1_of_x():
        z1bn = jnp.dot(xa_ref[...], w0f[...],
                       preferred_element_type=jnp.float32)
        return jnp.maximum(z1bn, 0.0).astype(jnp.bfloat16)

    @pl.when(p == 0)
    def _():
        z1 = jnp.dot(xa_ref[...], w0a_ref[...],
                     preferred_element_type=jnp.float32)
        _acc_stats(st1, z1, i)

        @pl.when(i == nb - 1)
        def _():
            _fold(st1, g0_ref, be0_ref, ss1, inv_n)
            scale = ss1[0:1, :]
            w0f[...] = (w0a_ref[...].astype(jnp.float32)
                        * scale).astype(jnp.bfloat16)
            w0f[w0a_ref.shape[0] - 1:w0a_ref.shape[0], :] = (
                ss1[1:2, :].astype(jnp.bfloat16))

    @pl.when(p == 1)
    def _():
        h1 = h1_of_x()
        z2 = jnp.dot(h1, w1_ref[...], preferred_element_type=jnp.float32)
        _acc_stats(st2, z2, i)

        @pl.when(i == nb - 1)
        def _():
            _fold(st2, g1_ref, be1_ref, ss2, inv_n)
            w1f[...] = (w1_ref[...].astype(jnp.float32)
                        * ss2[0:1, :]).astype(jnp.bfloat16)

    @pl.when(p == 2)
    def _():
        h1 = h1_of_x()
        z2bn = jnp.dot(h1, w1f[...],
                       preferred_element_type=jnp.float32) + ss2[1:2, :]
        h2 = jnp.maximum(z2bn, 0.0).astype(jnp.bfloat16)
        z3 = jnp.dot(h2, w2_ref[...], preferred_element_type=jnp.float32)
        _acc_stats(st3, z3, i)
        _store_rows(z3c, i * tm, z3.astype(jnp.bfloat16))

        @pl.when(i == nb - 1)
        def _():
            _fold(st3, g2_ref, be2_ref, ss3, inv_n)

    @pl.when(p == 3)
    def _():
        z3 = z3c[pl.ds(i * tm, tm), :].astype(jnp.float32)
        h3 = jnp.maximum(z3 * ss3[0:1, :] + ss3[1:2, :],
                         0.0).astype(jnp.bfloat16)
        _store_rows(z3c, i * tm, h3)
        z4 = jnp.dot(h3, w3_ref[...], preferred_element_type=jnp.float32)
        _acc_stats(st4, z4, i)

        @pl.when(i == nb - 1)
        def _():
            _fold(st4, g3_ref, be3_ref, ss4, inv_n)

    @pl.when(p == 4)
    def _():
        h3 = z3c[pl.ds(i * tm, tm), :]
        z4 = jnp.dot(h3, w3_ref[...], preferred_element_type=jnp.float32)
        o_ref[...] = z4 * ss4[0:1, :] + ss4[1:2, :]


def kernel(x, w0, b0, g0, be0, w1, b1, g1, be1, w2, b2, g2, be2,
           w3, b3, g3, be3):
    n, f_in = x.shape
    f1, f2, f3, f4 = w0.shape[1], w1.shape[1], w2.shape[1], w3.shape[1]
    tm = _pick_tm(n)
    nb = n // tm
    fa = f_in + 1

    xa = jnp.concatenate(
        [x, jnp.ones((n, 1), x.dtype)], axis=1).astype(jnp.bfloat16)
    w0a = jnp.concatenate(
        [w0, jnp.zeros((1, f1), w0.dtype)], axis=0).astype(jnp.bfloat16)
    w1b, w2b, w3b = (w.astype(jnp.bfloat16) for w in (w1, w2, w3))

    fixed = lambda p, i: (0, 0)
    body = functools.partial(_mlp_bn_body, tm, nb, 1.0 / n)

    return pl.pallas_call(
        body,
        out_shape=jax.ShapeDtypeStruct((n, f4), jnp.float32),
        grid=(5, nb),
        in_specs=[
            pl.BlockSpec((tm, fa), lambda p, i: (jnp.where(p < 3, i, 0), 0)),
            pl.BlockSpec((fa, f1), fixed),
            pl.BlockSpec((f1, f2), fixed),
            pl.BlockSpec((f2, f3), fixed),
            pl.BlockSpec((f3, f4), fixed),
            pl.BlockSpec((1, f1), fixed), pl.BlockSpec((1, f1), fixed),
            pl.BlockSpec((1, f2), fixed), pl.BlockSpec((1, f2), fixed),
            pl.BlockSpec((1, f3), fixed), pl.BlockSpec((1, f3), fixed),
            pl.BlockSpec((1, f4), fixed), pl.BlockSpec((1, f4), fixed),
        ],
        out_specs=pl.BlockSpec((tm, f4),
                               lambda p, i: (jnp.where(p == 4, i, 0), 0)),
        scratch_shapes=[
            pltpu.VMEM((n, f3), jnp.bfloat16),
            pltpu.VMEM((fa, f1), jnp.bfloat16),
            pltpu.VMEM((f1, f2), jnp.bfloat16),
            pltpu.VMEM((2, f1), jnp.float32),
            pltpu.VMEM((2, f2), jnp.float32),
            pltpu.VMEM((2, f3), jnp.float32),
            pltpu.VMEM((2, f4), jnp.float32),
            pltpu.VMEM((2, f1), jnp.float32),
            pltpu.VMEM((2, f2), jnp.float32),
            pltpu.VMEM((2, f3), jnp.float32),
            pltpu.VMEM((2, f4), jnp.float32),
        ],
        compiler_params=pltpu.CompilerParams(
            dimension_semantics=("arbitrary", "arbitrary"),
            vmem_limit_bytes=56 * 1024 * 1024,
        ),
        name="mlp_bn4_fused",
    )(xa, w0a, w1b, w2b, w3b, g0, be0, g1, be1, g2, be2, g3, be3)


# speedup vs baseline: 3.8250x; 1.1195x over previous
import functools

import jax
import jax.numpy as jnp
from jax.experimental import pallas as pl
from jax.experimental.pallas import tpu as pltpu

_EPS = 1e-5
_ROW_CHUNK = 2048


def _pick_tm(n):
    for t in (2048, 1024, 512, 256, 128, 64, 32, 16, 8):
        if n % t == 0:
            return t
    return n


def _acc_stats(st_ref, z, i):
    @pl.when(i == 0)
    def _():
        st_ref[...] = jnp.zeros_like(st_ref)

    ones = jnp.ones((1, z.shape[0]), jnp.float32)
    st_ref[0:1, :] += jnp.dot(ones, z, preferred_element_type=jnp.float32)
    st_ref[1:2, :] += jnp.dot(ones, z * z, preferred_element_type=jnp.float32)


def _fold(st_ref, g_ref, be_ref, ss_ref, inv_n):
    mu = st_ref[0:1, :] * inv_n
    var = st_ref[1:2, :] * inv_n - mu * mu
    scale = g_ref[...] * jax.lax.rsqrt(var + _EPS)
    ss_ref[0:1, :] = scale
    ss_ref[1:2, :] = be_ref[...] - mu * scale


def _store_rows(dst_ref, base, vals_bf16, lanes=None):
    tm = vals_bf16.shape[0]
    sl = slice(None) if lanes is None else slice(0, lanes)
    for r0 in range(0, tm, _ROW_CHUNK):
        r1 = min(r0 + _ROW_CHUNK, tm)
        dst_ref[pl.ds(base + r0, r1 - r0), sl] = vals_bf16[r0:r1, :]


def _mlp_bn_body(tm, nb, inv_n,
                 x_ref, w0_ref, w1_ref, w2_ref, w3_ref,
                 g0_ref, be0_ref, g1_ref, be1_ref,
                 g2_ref, be2_ref, g3_ref, be3_ref,
                 o_ref,
                 z2c, w0f, w2f, st1, st2, st3, st4, ss1, ss2, ss3, ss4):
    p = pl.program_id(0)
    i = pl.program_id(1)
    f3 = w2_ref.shape[1]

    def h2_of_cache():
        z2 = z2c[pl.ds(i * tm, tm), :].astype(jnp.float32)
        h2 = jnp.maximum(z2 * ss2[0:1, :] + ss2[1:2, :], 0.0)
        return h2.astype(jnp.bfloat16)

    @pl.when(p == 0)
    def _():
        z1 = jnp.dot(x_ref[...].astype(jnp.bfloat16), w0_ref[...],
                     preferred_element_type=jnp.float32)
        _acc_stats(st1, z1, i)

        @pl.when(i == nb - 1)
        def _():
            _fold(st1, g0_ref, be0_ref, ss1, inv_n)
            w0f[...] = (w0_ref[...].astype(jnp.float32)
                        * ss1[0:1, :]).astype(jnp.bfloat16)

    @pl.when(p == 1)
    def _():
        z1bn = jnp.dot(x_ref[...].astype(jnp.bfloat16), w0f[...],
                       preferred_element_type=jnp.float32) + ss1[1:2, :]
        h1 = jnp.maximum(z1bn, 0.0).astype(jnp.bfloat16)
        z2 = jnp.dot(h1, w1_ref[...], preferred_element_type=jnp.float32)
        _acc_stats(st2, z2, i)
        _store_rows(z2c, i * tm, z2.astype(jnp.bfloat16))

        @pl.when(i == nb - 1)
        def _():
            _fold(st2, g1_ref, be1_ref, ss2, inv_n)

    @pl.when(p == 2)
    def _():
        h2 = h2_of_cache()
        z3 = jnp.dot(h2, w2_ref[...], preferred_element_type=jnp.float32)
        _acc_stats(st3, z3, i)

        @pl.when(i == nb - 1)
        def _():
            _fold(st3, g2_ref, be2_ref, ss3, inv_n)
            w2f[...] = (w2_ref[...].astype(jnp.float32)
                        * ss3[0:1, :]).astype(jnp.bfloat16)

    @pl.when(p == 3)
    def _():
        h2 = h2_of_cache()
        z3bn = jnp.dot(h2, w2f[...],
                       preferred_element_type=jnp.float32) + ss3[1:2, :]
        h3 = jnp.maximum(z3bn, 0.0).astype(jnp.bfloat16)
        _store_rows(z2c, i * tm, h3, lanes=f3)
        z4 = jnp.dot(h3, w3_ref[...], preferred_element_type=jnp.float32)
        _acc_stats(st4, z4, i)

        @pl.when(i == nb - 1)
        def _():
            _fold(st4, g3_ref, be3_ref, ss4, inv_n)

    @pl.when(p == 4)
    def _():
        h3 = z2c[pl.ds(i * tm, tm), 0:f3]
        z4 = jnp.dot(h3, w3_ref[...], preferred_element_type=jnp.float32)
        o_ref[...] = z4 * ss4[0:1, :] + ss4[1:2, :]


def kernel(x, w0, b0, g0, be0, w1, b1, g1, be1, w2, b2, g2, be2,
           w3, b3, g3, be3):
    n, f_in = x.shape
    f1, f2, f3, f4 = w0.shape[1], w1.shape[1], w2.shape[1], w3.shape[1]
    tm = _pick_tm(n)
    nb = n // tm

    w0b, w1b, w2b, w3b = (w.astype(jnp.bfloat16) for w in (w0, w1, w2, w3))

    fixed = lambda p, i: (0, 0)
    body = functools.partial(_mlp_bn_body, tm, nb, 1.0 / n)

    return pl.pallas_call(
        body,
        out_shape=jax.ShapeDtypeStruct((n, f4), jnp.float32),
        grid=(5, nb),
        in_specs=[
            pl.BlockSpec((tm, f_in), lambda p, i: (jnp.where(p < 2, i, 0), 0)),
            pl.BlockSpec((f_in, f1), fixed),
            pl.BlockSpec((f1, f2), fixed),
            pl.BlockSpec((f2, f3), fixed),
            pl.BlockSpec((f3, f4), fixed),
            pl.BlockSpec((1, f1), fixed), pl.BlockSpec((1, f1), fixed),
            pl.BlockSpec((1, f2), fixed), pl.BlockSpec((1, f2), fixed),
            pl.BlockSpec((1, f3), fixed), pl.BlockSpec((1, f3), fixed),
            pl.BlockSpec((1, f4), fixed), pl.BlockSpec((1, f4), fixed),
        ],
        out_specs=pl.BlockSpec((tm, f4),
                               lambda p, i: (jnp.where(p == 4, i, 0), 0)),
        scratch_shapes=[
            pltpu.VMEM((n, f2), jnp.bfloat16),
            pltpu.VMEM((f_in, f1), jnp.bfloat16),
            pltpu.VMEM((f2, f3), jnp.bfloat16),
            pltpu.VMEM((2, f1), jnp.float32),
            pltpu.VMEM((2, f2), jnp.float32),
            pltpu.VMEM((2, f3), jnp.float32),
            pltpu.VMEM((2, f4), jnp.float32),
            pltpu.VMEM((2, f1), jnp.float32),
            pltpu.VMEM((2, f2), jnp.float32),
            pltpu.VMEM((2, f3), jnp.float32),
            pltpu.VMEM((2, f4), jnp.float32),
        ],
        compiler_params=pltpu.CompilerParams(
            dimension_semantics=("arbitrary", "arbitrary"),
            vmem_limit_bytes=56 * 1024 * 1024,
        ),
        name="mlp_bn4_fused",
    )(x, w0b, w1b, w2b, w3b, g0, be0, g1, be1, g2, be2, g3, be3)


# speedup vs baseline: 4.4107x; 1.1531x over previous
import functools

import jax
import jax.numpy as jnp
from jax.experimental import pallas as pl
from jax.experimental.pallas import tpu as pltpu

_EPS = 1e-5
_ROW_CHUNK = 2048


def _pick_tm(n):
    for t in (4096, 2048, 1024, 512, 256, 128, 64, 32, 16, 8):
        if n % t == 0:
            return t
    return n


def _acc_stats(st_ref, z):
    ones = jnp.ones((1, z.shape[0]), jnp.float32)
    st_ref[0:1, :] += jnp.dot(ones, z, preferred_element_type=jnp.float32)
    st_ref[1:2, :] += jnp.dot(ones, z * z, preferred_element_type=jnp.float32)


def _fold(st_ref, g_ref, be_ref, ss_ref, inv_n):
    mu = st_ref[0:1, :] * inv_n
    var = st_ref[1:2, :] * inv_n - mu * mu
    scale = g_ref[...] * jax.lax.rsqrt(var + _EPS)
    ss_ref[0:1, :] = scale
    ss_ref[1:2, :] = be_ref[...] - mu * scale


def _mlp_bn_body(tm, nb, inv_n,
                 x_ref, w0_ref, w1_ref, w2_ref, w3_ref,
                 g0_ref, be0_ref, g1_ref, be1_ref,
                 g2_ref, be2_ref, g3_ref, be3_ref,
                 o_ref,
                 z2c, w0f, w2f, st1, st2, st3, st4, ss1, ss2, ss3, ss4):
    p = pl.program_id(0)
    i = pl.program_id(1)
    f3 = w2_ref.shape[1]
    cz = min(_ROW_CHUNK, tm)
    chunks = range(0, tm, cz)

    @pl.when(p == 0)
    def _():
        @pl.when(i == 0)
        def _():
            st1[...] = jnp.zeros_like(st1)

        for r in chunks:
            xb = x_ref[pl.ds(r, cz), :].astype(jnp.bfloat16)
            z1 = jnp.dot(xb, w0_ref[...], preferred_element_type=jnp.float32)
            _acc_stats(st1, z1)

        @pl.when(i == nb - 1)
        def _():
            _fold(st1, g0_ref, be0_ref, ss1, inv_n)
            w0f[...] = (w0_ref[...].astype(jnp.float32)
                        * ss1[0:1, :]).astype(jnp.bfloat16)

    @pl.when(p == 1)
    def _():
        @pl.when(i == 0)
        def _():
            st2[...] = jnp.zeros_like(st2)

        for r in chunks:
            xb = x_ref[pl.ds(r, cz), :].astype(jnp.bfloat16)
            z1bn = jnp.dot(xb, w0f[...],
                           preferred_element_type=jnp.float32) + ss1[1:2, :]
            h1 = jnp.maximum(z1bn, 0.0).astype(jnp.bfloat16)
            z2 = jnp.dot(h1, w1_ref[...], preferred_element_type=jnp.float32)
            _acc_stats(st2, z2)
            z2c[pl.ds(i * tm + r, cz), :] = z2.astype(jnp.bfloat16)

        @pl.when(i == nb - 1)
        def _():
            _fold(st2, g1_ref, be1_ref, ss2, inv_n)

    @pl.when(p == 2)
    def _():
        @pl.when(i == 0)
        def _():
            st3[...] = jnp.zeros_like(st3)

        for r in chunks:
            z2 = z2c[pl.ds(i * tm + r, cz), :].astype(jnp.float32)
            h2 = jnp.maximum(z2 * ss2[0:1, :] + ss2[1:2, :], 0.0)
            h2 = h2.astype(jnp.bfloat16)
            z2c[pl.ds(i * tm + r, cz), :] = h2
            z3 = jnp.dot(h2, w2_ref[...], preferred_element_type=jnp.float32)
            _acc_stats(st3, z3)

        @pl.when(i == nb - 1)
        def _():
            _fold(st3, g2_ref, be2_ref, ss3, inv_n)
            w2f[...] = (w2_ref[...].astype(jnp.float32)
                        * ss3[0:1, :]).astype(jnp.bfloat16)

    @pl.when(p == 3)
    def _():
        @pl.when(i == 0)
        def _():
            st4[...] = jnp.zeros_like(st4)

        for r in chunks:
            h2 = z2c[pl.ds(i * tm + r, cz), :]
            z3bn = jnp.dot(h2, w2f[...],
                           preferred_element_type=jnp.float32) + ss3[1:2, :]
            h3 = jnp.maximum(z3bn, 0.0).astype(jnp.bfloat16)
            z2c[pl.ds(i * tm + r, cz), 0:f3] = h3
            z4 = jnp.dot(h3, w3_ref[...], preferred_element_type=jnp.float32)
            _acc_stats(st4, z4)

        @pl.when(i == nb - 1)
        def _():
            _fold(st4, g3_ref, be3_ref, ss4, inv_n)

    @pl.when(p == 4)
    def _():
        for r in chunks:
            h3 = z2c[pl.ds(i * tm + r, cz), 0:f3]
            z4 = jnp.dot(h3, w3_ref[...], preferred_element_type=jnp.float32)
            o_ref[pl.ds(r, cz), :] = z4 * ss4[0:1, :] + ss4[1:2, :]


def kernel(x, w0, b0, g0, be0, w1, b1, g1, be1, w2, b2, g2, be2,
           w3, b3, g3, be3):
    n, f_in = x.shape
    f1, f2, f3, f4 = w0.shape[1], w1.shape[1], w2.shape[1], w3.shape[1]
    tm = _pick_tm(n)
    nb = n // tm

    w0b, w1b, w2b, w3b = (w.astype(jnp.bfloat16) for w in (w0, w1, w2, w3))

    fixed = lambda p, i: (0, 0)
    body = functools.partial(_mlp_bn_body, tm, nb, 1.0 / n)

    return pl.pallas_call(
        body,
        out_shape=jax.ShapeDtypeStruct((n, f4), jnp.float32),
        grid=(5, nb),
        in_specs=[
            pl.BlockSpec((tm, f_in), lambda p, i: (jnp.where(p < 2, i, 0), 0)),
            pl.BlockSpec((f_in, f1), fixed),
            pl.BlockSpec((f1, f2), fixed),
            pl.BlockSpec((f2, f3), fixed),
            pl.BlockSpec((f3, f4), fixed),
            pl.BlockSpec((1, f1), fixed), pl.BlockSpec((1, f1), fixed),
            pl.BlockSpec((1, f2), fixed), pl.BlockSpec((1, f2), fixed),
            pl.BlockSpec((1, f3), fixed), pl.BlockSpec((1, f3), fixed),
            pl.BlockSpec((1, f4), fixed), pl.BlockSpec((1, f4), fixed),
        ],
        out_specs=pl.BlockSpec((tm, f4),
                               lambda p, i: (jnp.where(p == 4, i, 0), 0)),
        scratch_shapes=[
            pltpu.VMEM((n, f2), jnp.bfloat16),
            pltpu.VMEM((f_in, f1), jnp.bfloat16),
            pltpu.VMEM((f2, f3), jnp.bfloat16),
            pltpu.VMEM((2, f1), jnp.float32),
            pltpu.VMEM((2, f2), jnp.float32),
            pltpu.VMEM((2, f3), jnp.float32),
            pltpu.VMEM((2, f4), jnp.float32),
            pltpu.VMEM((2, f1), jnp.float32),
            pltpu.VMEM((2, f2), jnp.float32),
            pltpu.VMEM((2, f3), jnp.float32),
            pltpu.VMEM((2, f4), jnp.float32),
        ],
        compiler_params=pltpu.CompilerParams(
            dimension_semantics=("arbitrary", "arbitrary"),
            vmem_limit_bytes=60000 * 1024,
        ),
        name="mlp_bn4_fused",
    )(x, w0b, w1b, w2b, w3b, g0, be0, g1, be1, g2, be2, g3, be3)


# speedup vs baseline: 4.5567x; 1.0331x over previous
import functools

import jax
import jax.numpy as jnp
from jax.experimental import pallas as pl
from jax.experimental.pallas import tpu as pltpu

_EPS = 1e-5
_ROW_CHUNK = 2048


def _pick_tm(n):
    for t in (4096, 2048, 1024, 512, 256, 128, 64, 32, 16, 8):
        if n % t == 0:
            return t
    return n


def _acc_stats(st_ref, z):
    ones = jnp.ones((1, z.shape[0]), jnp.float32)
    st_ref[0:1, :] += jnp.dot(ones, z, preferred_element_type=jnp.float32)
    st_ref[1:2, :] += jnp.dot(ones, z * z, preferred_element_type=jnp.float32)


def _fold(st_ref, g_ref, be_ref, ss_ref, inv_n):
    mu = st_ref[0:1, :] * inv_n
    var = st_ref[1:2, :] * inv_n - mu * mu
    scale = g_ref[...] * jax.lax.rsqrt(var + _EPS)
    ss_ref[0:1, :] = scale
    ss_ref[1:2, :] = be_ref[...] - mu * scale


def _mlp_bn_body(tm, nb, inv_n,
                 x_ref, w0_ref, w1_ref, w2_ref, w3_ref,
                 g0_ref, be0_ref, g1_ref, be1_ref,
                 g2_ref, be2_ref, g3_ref, be3_ref,
                 o_ref,
                 z2c, w0f, st1, st2, st3, st4, ss1, ss2, ss3, ss4):
    p = pl.program_id(0)
    i = pl.program_id(1)
    f3 = w2_ref.shape[1]
    cz = min(_ROW_CHUNK, tm)
    chunks = range(0, tm, cz)

    def xb_aug(r):
        xb = x_ref[pl.ds(r, cz), :].astype(jnp.bfloat16)
        return jnp.concatenate(
            [xb, jnp.ones((cz, 1), jnp.bfloat16)], axis=1)

    @pl.when(p == 0)
    def _():
        @pl.when(i == 0)
        def _():
            st1[...] = jnp.zeros_like(st1)
            w0f[0:w0_ref.shape[0], :] = w0_ref[...]
            w0f[w0_ref.shape[0]:w0_ref.shape[0] + 1, :] = jnp.zeros(
                (1, w0_ref.shape[1]), jnp.bfloat16)

        for r in chunks:
            z1 = jnp.dot(xb_aug(r), w0f[...],
                         preferred_element_type=jnp.float32)
            _acc_stats(st1, z1)

        @pl.when(i == nb - 1)
        def _():
            _fold(st1, g0_ref, be0_ref, ss1, inv_n)
            w0f[0:w0_ref.shape[0], :] = (
                w0_ref[...].astype(jnp.float32)
                * ss1[0:1, :]).astype(jnp.bfloat16)
            w0f[w0_ref.shape[0]:w0_ref.shape[0] + 1, :] = (
                ss1[1:2, :].astype(jnp.bfloat16))

    @pl.when(p == 1)
    def _():
        @pl.when(i == 0)
        def _():
            st2[...] = jnp.zeros_like(st2)

        for r in chunks:
            z1bn = jnp.dot(xb_aug(r), w0f[...],
                           preferred_element_type=jnp.float32)
            h1 = jnp.maximum(z1bn, 0.0).astype(jnp.bfloat16)
            z2 = jnp.dot(h1, w1_ref[...], preferred_element_type=jnp.float32)
            _acc_stats(st2, z2)
            z2c[pl.ds(i * tm + r, cz), :] = z2.astype(jnp.bfloat16)

        @pl.when(i == nb - 1)
        def _():
            _fold(st2, g1_ref, be1_ref, ss2, inv_n)

    @pl.when(p == 2)
    def _():
        @pl.when(i == 0)
        def _():
            st3[...] = jnp.zeros_like(st3)

        for r in chunks:
            z2 = z2c[pl.ds(i * tm + r, cz), :].astype(jnp.float32)
            h2 = jnp.maximum(z2 * ss2[0:1, :] + ss2[1:2, :], 0.0)
            h2 = h2.astype(jnp.bfloat16)
            z3 = jnp.dot(h2, w2_ref[...], preferred_element_type=jnp.float32)
            _acc_stats(st3, z3)
            z2c[pl.ds(i * tm + r, cz), 0:f3] = z3.astype(jnp.bfloat16)

        @pl.when(i == nb - 1)
        def _():
            _fold(st3, g2_ref, be2_ref, ss3, inv_n)

    @pl.when(p == 3)
    def _():
        @pl.when(i == 0)
        def _():
            st4[...] = jnp.zeros_like(st4)

        for r in chunks:
            z3 = z2c[pl.ds(i * tm + r, cz), 0:f3].astype(jnp.float32)
            h3 = jnp.maximum(z3 * ss3[0:1, :] + ss3[1:2, :],
                             0.0).astype(jnp.bfloat16)
            z2c[pl.ds(i * tm + r, cz), 0:f3] = h3
            z4 = jnp.dot(h3, w3_ref[...], preferred_element_type=jnp.float32)
            _acc_stats(st4, z4)

        @pl.when(i == nb - 1)
        def _():
            _fold(st4, g3_ref, be3_ref, ss4, inv_n)

    @pl.when(p == 4)
    def _():
        for r in chunks:
            h3 = z2c[pl.ds(i * tm + r, cz), 0:f3]
            z4 = jnp.dot(h3, w3_ref[...], preferred_element_type=jnp.float32)
            o_ref[pl.ds(r, cz), :] = z4 * ss4[0:1, :] + ss4[1:2, :]


def kernel(x, w0, b0, g0, be0, w1, b1, g1, be1, w2, b2, g2, be2,
           w3, b3, g3, be3):
    n, f_in = x.shape
    f1, f2, f3, f4 = w0.shape[1], w1.shape[1], w2.shape[1], w3.shape[1]
    tm = _pick_tm(n)
    nb = n // tm

    w0b, w1b, w2b, w3b = (w.astype(jnp.bfloat16) for w in (w0, w1, w2, w3))

    fixed = lambda p, i: (0, 0)
    body = functools.partial(_mlp_bn_body, tm, nb, 1.0 / n)

    return pl.pallas_call(
        body,
        out_shape=jax.ShapeDtypeStruct((n, f4), jnp.float32),
        grid=(5, nb),
        in_specs=[
            pl.BlockSpec((tm, f_in), lambda p, i: (jnp.where(p < 2, i, 0), 0)),
            pl.BlockSpec((f_in, f1), fixed),
            pl.BlockSpec((f1, f2), fixed),
            pl.BlockSpec((f2, f3), fixed),
            pl.BlockSpec((f3, f4), fixed),
            pl.BlockSpec((1, f1), fixed), pl.BlockSpec((1, f1), fixed),
            pl.BlockSpec((1, f2), fixed), pl.BlockSpec((1, f2), fixed),
            pl.BlockSpec((1, f3), fixed), pl.BlockSpec((1, f3), fixed),
            pl.BlockSpec((1, f4), fixed), pl.BlockSpec((1, f4), fixed),
        ],
        out_specs=pl.BlockSpec((tm, f4),
                               lambda p, i: (jnp.where(p == 4, i, 0), 0)),
        scratch_shapes=[
            pltpu.VMEM((n, f2), jnp.bfloat16),
            pltpu.VMEM((f_in + 1, f1), jnp.bfloat16),
            pltpu.VMEM((2, f1), jnp.float32),
            pltpu.VMEM((2, f2), jnp.float32),
            pltpu.VMEM((2, f3), jnp.float32),
            pltpu.VMEM((2, f4), jnp.float32),
            pltpu.VMEM((2, f1), jnp.float32),
            pltpu.VMEM((2, f2), jnp.float32),
            pltpu.VMEM((2, f3), jnp.float32),
            pltpu.VMEM((2, f4), jnp.float32),
        ],
        compiler_params=pltpu.CompilerParams(
            dimension_semantics=("arbitrary", "arbitrary"),
            vmem_limit_bytes=60000 * 1024,
        ),
        name="mlp_bn4_fused",
    )(x, w0b, w1b, w2b, w3b, g0, be0, g1, be1, g2, be2, g3, be3)
